# Initial kernel scaffold; baseline (speedup 1.0000x reference)
import jax
import jax.numpy as jnp
from jax.experimental import pallas as pl


def kernel(emb_src, emb_tgt, W_enc, b_enc, W_dec, src_id, tgt_id):
    raise NotImplementedError("write your pallas kernel here")


# speedup vs baseline: 2.3689x; 2.3689x over previous
import functools

import jax
import jax.numpy as jnp
from jax import lax
from jax.experimental import pallas as pl
from jax.experimental.pallas import tpu as pltpu
from jax.experimental.pallas import tpu_sc as plsc

DIM = 300
BATCH = 16384

_NC, _NS = 2, 16
NW = _NC * _NS
ROWS_PER_W = BATCH // NW

BLK = 16


@functools.cache
def _sc_gather_fn():
    mesh = plsc.VectorSubcoreMesh(core_axis_name="c", subcore_axis_name="s")

    @functools.partial(
        pl.kernel,
        mesh=mesh,
        out_type=[
            jax.ShapeDtypeStruct((BATCH, DIM), jnp.float32),
            jax.ShapeDtypeStruct((BATCH, DIM), jnp.float32),
        ],
        scratch_types=[
            pltpu.VMEM((ROWS_PER_W,), jnp.int32),
            pltpu.VMEM((ROWS_PER_W,), jnp.int32),
            pltpu.VMEM((BLK, DIM), jnp.float32),
            pltpu.VMEM((BLK, DIM), jnp.float32),
            pltpu.SemaphoreType.DMA,
            pltpu.SemaphoreType.DMA,
        ],
    )
    def _sc_gather(src_tab, tgt_tab, src_idx, tgt_idx, src_out, tgt_out,
                   sidx_s, tidx_s, buf0, buf1, sem0, sem1):
        wid = lax.axis_index("s") * _NC + lax.axis_index("c")
        base = wid * ROWS_PER_W
        pltpu.sync_copy(src_idx.at[wid], sidx_s)
        pltpu.sync_copy(tgt_idx.at[wid], tidx_s)
        bufs = (buf0, buf1)
        sems = (sem0, sem1)
        for idx_s, tab, out in ((sidx_s, src_tab, src_out),
                                (tidx_s, tgt_tab, tgt_out)):
            @pl.loop(0, ROWS_PER_W // BLK, step=2)
            def _(o):
                descs = []
                for half in range(2):
                    vec = idx_s[pl.ds((o + half) * BLK, BLK)]
                    for j in range(BLK):
                        descs.append(pltpu.async_copy(
                            tab.at[vec[j]], bufs[half].at[j], sems[half]))
                for half in range(2):
                    for d in descs[half * BLK:(half + 1) * BLK]:
                        d.wait()
                    pltpu.sync_copy(
                        bufs[half],
                        out.at[pl.ds(base + (o + half) * BLK, BLK)])

    return _sc_gather


BM = 1024


def _mm_body(x_ref, we_ref, wd_ref, b_ref, o_ref, m_ref, bv_ref):
    @pl.when(pl.program_id(0) == 0)
    def _():
        m_ref[...] = lax.dot_general(
            we_ref[...], wd_ref[...],
            dimension_numbers=(((0,), (0,)), ((), ())),
            preferred_element_type=jnp.float32)
        bv_ref[...] = jnp.dot(b_ref[...], wd_ref[...],
                              preferred_element_type=jnp.float32)

    o_ref[...] = jnp.dot(x_ref[...], m_ref[...],
                         preferred_element_type=jnp.float32) + bv_ref[...]


_mm = pl.pallas_call(
    _mm_body,
    grid=(BATCH // BM,),
    in_specs=[
        pl.BlockSpec((BM, DIM), lambda i: (i, 0)),
        pl.BlockSpec((DIM, DIM), lambda i: (0, 0)),
        pl.BlockSpec((DIM, DIM), lambda i: (0, 0)),
        pl.BlockSpec((1, DIM), lambda i: (0, 0)),
    ],
    out_specs=pl.BlockSpec((BM, DIM), lambda i: (i, 0)),
    out_shape=jax.ShapeDtypeStruct((BATCH, DIM), jnp.float32),
    scratch_shapes=[
        pltpu.VMEM((DIM, DIM), jnp.float32),
        pltpu.VMEM((1, DIM), jnp.float32),
    ],
)


def kernel(emb_src, emb_tgt, W_enc, b_enc, W_dec, src_id, tgt_id):
    sidx = src_id.astype(jnp.int32).reshape(NW, ROWS_PER_W)
    tidx = tgt_id.astype(jnp.int32).reshape(NW, ROWS_PER_W)
    src_rows, tgt_rows = _sc_gather_fn()(emb_src, emb_tgt, sidx, tidx)
    top = _mm(src_rows, W_enc, W_dec, b_enc.reshape(1, DIM))
    return jnp.concatenate([top, tgt_rows], axis=0)


# speedup vs baseline: 2.5573x; 1.0795x over previous
import functools

import jax
import jax.numpy as jnp
from jax import lax
from jax.experimental import pallas as pl
from jax.experimental.pallas import tpu as pltpu
from jax.experimental.pallas import tpu_sc as plsc

DIM = 300
BATCH = 16384

_NC, _NS = 2, 16
NW = _NC * _NS
ROWS_PER_W = BATCH // NW

BLK = 16


@functools.cache
def _sc_gather_fn():
    mesh = plsc.VectorSubcoreMesh(core_axis_name="c", subcore_axis_name="s")

    @functools.partial(
        pl.kernel,
        mesh=mesh,
        out_type=[
            jax.ShapeDtypeStruct((BATCH, DIM), jnp.float32),
            jax.ShapeDtypeStruct((2 * BATCH, DIM), jnp.float32),
        ],
        scratch_types=[
            pltpu.VMEM((ROWS_PER_W,), jnp.int32),
            pltpu.VMEM((ROWS_PER_W,), jnp.int32),
            pltpu.VMEM((BLK, DIM), jnp.float32),
            pltpu.VMEM((BLK, DIM), jnp.float32),
            pltpu.SemaphoreType.DMA,
            pltpu.SemaphoreType.DMA,
        ],
    )
    def _sc_gather(src_tab, tgt_tab, src_idx, tgt_idx, src_out, full_out,
                   sidx_s, tidx_s, buf0, buf1, sem0, sem1):
        wid = lax.axis_index("s") * _NC + lax.axis_index("c")
        base = wid * ROWS_PER_W
        pltpu.sync_copy(src_idx.at[wid], sidx_s)
        pltpu.sync_copy(tgt_idx.at[wid], tidx_s)
        bufs = (buf0, buf1)
        sems = (sem0, sem1)
        for idx_s, tab, out, obase in ((sidx_s, src_tab, src_out, base),
                                       (tidx_s, tgt_tab, full_out,
                                        BATCH + base)):
            @pl.loop(0, ROWS_PER_W // BLK, step=2)
            def _(o):
                descs = []
                for half in range(2):
                    vec = idx_s[pl.ds((o + half) * BLK, BLK)]
                    for j in range(BLK):
                        descs.append(pltpu.async_copy(
                            tab.at[vec[j]], bufs[half].at[j], sems[half]))
                for half in range(2):
                    for d in descs[half * BLK:(half + 1) * BLK]:
                        d.wait()
                    pltpu.sync_copy(
                        bufs[half],
                        out.at[pl.ds(obase + (o + half) * BLK, BLK)])

    return _sc_gather


BM = 1024


def _mm_body(x_ref, we_ref, wd_ref, b_ref, full_ref, o_ref, m_ref, bv_ref):
    del full_ref
    @pl.when(pl.program_id(0) == 0)
    def _():
        m_ref[...] = lax.dot_general(
            we_ref[...], wd_ref[...],
            dimension_numbers=(((0,), (0,)), ((), ())),
            preferred_element_type=jnp.float32)
        bv_ref[...] = jnp.dot(b_ref[...], wd_ref[...],
                              preferred_element_type=jnp.float32)

    o_ref[...] = jnp.dot(x_ref[...], m_ref[...],
                         preferred_element_type=jnp.float32) + bv_ref[...]


_mm = pl.pallas_call(
    _mm_body,
    grid=(BATCH // BM,),
    in_specs=[
        pl.BlockSpec((BM, DIM), lambda i: (i, 0)),
        pl.BlockSpec((DIM, DIM), lambda i: (0, 0)),
        pl.BlockSpec((DIM, DIM), lambda i: (0, 0)),
        pl.BlockSpec((1, DIM), lambda i: (0, 0)),
        pl.BlockSpec(memory_space=pltpu.MemorySpace.HBM),
    ],
    out_specs=pl.BlockSpec((BM, DIM), lambda i: (i, 0)),
    out_shape=jax.ShapeDtypeStruct((2 * BATCH, DIM), jnp.float32),
    input_output_aliases={4: 0},
    scratch_shapes=[
        pltpu.VMEM((DIM, DIM), jnp.float32),
        pltpu.VMEM((1, DIM), jnp.float32),
    ],
)


def kernel(emb_src, emb_tgt, W_enc, b_enc, W_dec, src_id, tgt_id):
    sidx = src_id.astype(jnp.int32).reshape(NW, ROWS_PER_W)
    tidx = tgt_id.astype(jnp.int32).reshape(NW, ROWS_PER_W)
    src_rows, full = _sc_gather_fn()(emb_src, emb_tgt, sidx, tidx)
    return _mm(src_rows, W_enc, W_dec, b_enc.reshape(1, DIM), full)


# speedup vs baseline: 2.7051x; 1.0578x over previous
import functools

import jax
import jax.numpy as jnp
from jax import lax
from jax.experimental import pallas as pl
from jax.experimental.pallas import tpu as pltpu
from jax.experimental.pallas import tpu_sc as plsc

DIM = 300
BATCH = 16384
VOCAB = 75000

_NC, _NS = 2, 16
NW = _NC * _NS
ROWS_PER_W = BATCH // NW

BLK = 16

BMV = 512
VPAD = 75264
NVBLK = VPAD // BMV


def _tf_body(src_t_ref, tgt_t_ref, we_ref, wd_ref, b_ref,
             ts_ref, tt_ref, m_ref, bv_ref):
    @pl.when(pl.program_id(0) == 0)
    def _():
        m_ref[...] = lax.dot_general(
            we_ref[...], wd_ref[...],
            dimension_numbers=(((0,), (0,)), ((), ())),
            preferred_element_type=jnp.float32)
        bv_ref[...] = jnp.dot(b_ref[...], wd_ref[...],
                              preferred_element_type=jnp.float32)

    ts_ref[...] = lax.dot_general(
        src_t_ref[...], m_ref[...],
        dimension_numbers=(((0,), (0,)), ((), ())),
        preferred_element_type=jnp.float32) + bv_ref[...]
    tt_ref[...] = tgt_t_ref[...].T


_tf = pl.pallas_call(
    _tf_body,
    grid=(NVBLK,),
    in_specs=[
        pl.BlockSpec((DIM, BMV), lambda i: (0, i)),
        pl.BlockSpec((DIM, BMV), lambda i: (0, i)),
        pl.BlockSpec((DIM, DIM), lambda i: (0, 0)),
        pl.BlockSpec((DIM, DIM), lambda i: (0, 0)),
        pl.BlockSpec((1, DIM), lambda i: (0, 0)),
    ],
    out_specs=[
        pl.BlockSpec((BMV, DIM), lambda i: (i, 0)),
        pl.BlockSpec((BMV, DIM), lambda i: (i, 0)),
    ],
    out_shape=[
        jax.ShapeDtypeStruct((VPAD, DIM), jnp.float32),
        jax.ShapeDtypeStruct((VPAD, DIM), jnp.float32),
    ],
    scratch_shapes=[
        pltpu.VMEM((DIM, DIM), jnp.float32),
        pltpu.VMEM((1, DIM), jnp.float32),
    ],
)


@functools.cache
def _sc_gather_fn():
    mesh = plsc.VectorSubcoreMesh(core_axis_name="c", subcore_axis_name="s")

    @functools.partial(
        pl.kernel,
        mesh=mesh,
        out_type=jax.ShapeDtypeStruct((2 * BATCH, DIM), jnp.float32),
        scratch_types=[
            pltpu.VMEM((ROWS_PER_W,), jnp.int32),
            pltpu.VMEM((ROWS_PER_W,), jnp.int32),
            pltpu.VMEM((BLK, DIM), jnp.float32),
            pltpu.VMEM((BLK, DIM), jnp.float32),
            pltpu.SemaphoreType.DMA,
            pltpu.SemaphoreType.DMA,
        ],
    )
    def _sc_gather(src_tab, tgt_tab, src_idx, tgt_idx, full_out,
                   sidx_s, tidx_s, buf0, buf1, sem0, sem1):
        wid = lax.axis_index("s") * _NC + lax.axis_index("c")
        base = wid * ROWS_PER_W
        pltpu.sync_copy(src_idx.at[wid], sidx_s)
        pltpu.sync_copy(tgt_idx.at[wid], tidx_s)
        bufs = (buf0, buf1)
        sems = (sem0, sem1)
        for idx_s, tab, obase in ((sidx_s, src_tab, base),
                                  (tidx_s, tgt_tab, BATCH + base)):
            @pl.loop(0, ROWS_PER_W // BLK, step=2)
            def _(o):
                descs = []
                for half in range(2):
                    vec = idx_s[pl.ds((o + half) * BLK, BLK)]
                    for j in range(BLK):
                        descs.append(pltpu.async_copy(
                            tab.at[vec[j]], bufs[half].at[j], sems[half]))
                for half in range(2):
                    for d in descs[half * BLK:(half + 1) * BLK]:
                        d.wait()
                    pltpu.sync_copy(
                        bufs[half],
                        full_out.at[pl.ds(obase + (o + half) * BLK, BLK)])

    return _sc_gather


def kernel(emb_src, emb_tgt, W_enc, b_enc, W_dec, src_id, tgt_id):
    sidx = src_id.astype(jnp.int32).reshape(NW, ROWS_PER_W)
    tidx = tgt_id.astype(jnp.int32).reshape(NW, ROWS_PER_W)
    t_src, t_tgt = _tf(jnp.transpose(emb_src), jnp.transpose(emb_tgt),
                       W_enc, W_dec, b_enc.reshape(1, DIM))
    return _sc_gather_fn()(t_src, t_tgt, sidx, tidx)


# speedup vs baseline: 2.8704x; 1.0611x over previous
import functools

import jax
import jax.numpy as jnp
from jax import lax
from jax.experimental import pallas as pl
from jax.experimental.pallas import tpu as pltpu
from jax.experimental.pallas import tpu_sc as plsc

DIM = 300
BATCH = 16384
VOCAB = 75000

_NC, _NS = 2, 16
NW = _NC * _NS
ROWS_PER_W = BATCH // NW

BLK = 16

BMV = 1024
VPAD = 75776
NVBLK = VPAD // BMV

BM = 1024


def _eye_into(ref):
    r = lax.broadcasted_iota(jnp.int32, (DIM, DIM), 0)
    c = lax.broadcasted_iota(jnp.int32, (DIM, DIM), 1)
    ref[...] = (r == c).astype(jnp.float32)


def _tf_src_body(src_t_ref, we_ref, wd_ref, b_ref, ts_ref, m_ref, bv_ref):
    @pl.when(pl.program_id(0) == 0)
    def _():
        m_ref[...] = lax.dot_general(
            we_ref[...], wd_ref[...],
            dimension_numbers=(((0,), (0,)), ((), ())),
            preferred_element_type=jnp.float32)
        bv_ref[...] = jnp.dot(b_ref[...], wd_ref[...],
                              preferred_element_type=jnp.float32)

    ts_ref[...] = lax.dot_general(
        src_t_ref[...], m_ref[...],
        dimension_numbers=(((0,), (0,)), ((), ())),
        preferred_element_type=jnp.float32) + bv_ref[...]


_tf_src = pl.pallas_call(
    _tf_src_body,
    grid=(NVBLK,),
    in_specs=[
        pl.BlockSpec((DIM, BMV), lambda i: (0, i)),
        pl.BlockSpec((DIM, DIM), lambda i: (0, 0)),
        pl.BlockSpec((DIM, DIM), lambda i: (0, 0)),
        pl.BlockSpec((1, DIM), lambda i: (0, 0)),
    ],
    out_specs=pl.BlockSpec((BMV, DIM), lambda i: (i, 0)),
    out_shape=jax.ShapeDtypeStruct((VPAD, DIM), jnp.float32),
    scratch_shapes=[
        pltpu.VMEM((DIM, DIM), jnp.float32),
        pltpu.VMEM((1, DIM), jnp.float32),
    ],
)


def _tf_tgt_body(tgt_t_ref, tt_ref, eye_ref):
    @pl.when(pl.program_id(0) == 0)
    def _():
        _eye_into(eye_ref)

    tt_ref[...] = lax.dot_general(
        tgt_t_ref[...], eye_ref[...],
        dimension_numbers=(((0,), (0,)), ((), ())),
        preferred_element_type=jnp.float32)


_tf_tgt = pl.pallas_call(
    _tf_tgt_body,
    grid=(NVBLK,),
    in_specs=[pl.BlockSpec((DIM, BMV), lambda i: (0, i))],
    out_specs=pl.BlockSpec((BMV, DIM), lambda i: (i, 0)),
    out_shape=jax.ShapeDtypeStruct((VPAD, DIM), jnp.float32),
    scratch_shapes=[pltpu.VMEM((DIM, DIM), jnp.float32)],
)


@functools.cache
def _sc_gather_fn():
    mesh = plsc.VectorSubcoreMesh(core_axis_name="c", subcore_axis_name="s")

    @functools.partial(
        pl.kernel,
        mesh=mesh,
        out_type=jax.ShapeDtypeStruct((BATCH, DIM), jnp.float32),
        scratch_types=[
            pltpu.VMEM((ROWS_PER_W,), jnp.int32),
            pltpu.VMEM((BLK, DIM), jnp.float32),
            pltpu.VMEM((BLK, DIM), jnp.float32),
            pltpu.SemaphoreType.DMA,
            pltpu.SemaphoreType.DMA,
        ],
    )
    def _sc_gather(tab, idx, rows_out, idx_s, buf0, buf1, sem0, sem1):
        wid = lax.axis_index("s") * _NC + lax.axis_index("c")
        base = wid * ROWS_PER_W
        pltpu.sync_copy(idx.at[wid], idx_s)
        bufs = (buf0, buf1)
        sems = (sem0, sem1)

        @pl.loop(0, ROWS_PER_W // BLK, step=2)
        def _(o):
            descs = []
            for half in range(2):
                vec = idx_s[pl.ds((o + half) * BLK, BLK)]
                for j in range(BLK):
                    descs.append(pltpu.async_copy(
                        tab.at[vec[j]], bufs[half].at[j], sems[half]))
            for half in range(2):
                for d in descs[half * BLK:(half + 1) * BLK]:
                    d.wait()
                pltpu.sync_copy(
                    bufs[half],
                    rows_out.at[pl.ds(base + (o + half) * BLK, BLK)])

    return _sc_gather


def _trans_a_body(rows_ref, ot_ref, eye_ref):
    @pl.when(pl.program_id(0) == 0)
    def _():
        _eye_into(eye_ref)

    ot_ref[...] = lax.dot_general(
        eye_ref[...], rows_ref[...],
        dimension_numbers=(((0,), (1,)), ((), ())),
        preferred_element_type=jnp.float32)


def _trans_b_body(rows_ref, full_ref, ot_ref, eye_ref):
    del full_ref
    _trans_a_body(rows_ref, ot_ref, eye_ref)


_trans_a = pl.pallas_call(
    _trans_a_body,
    grid=(BATCH // BM,),
    in_specs=[pl.BlockSpec((BM, DIM), lambda i: (i, 0))],
    out_specs=pl.BlockSpec((DIM, BM), lambda i: (0, i)),
    out_shape=jax.ShapeDtypeStruct((DIM, 2 * BATCH), jnp.float32),
    scratch_shapes=[pltpu.VMEM((DIM, DIM), jnp.float32)],
)

_trans_b = pl.pallas_call(
    _trans_b_body,
    grid=(BATCH // BM,),
    in_specs=[
        pl.BlockSpec((BM, DIM), lambda i: (i, 0)),
        pl.BlockSpec(memory_space=pltpu.MemorySpace.HBM),
    ],
    out_specs=pl.BlockSpec((DIM, BM), lambda i: (0, i + BATCH // BM)),
    out_shape=jax.ShapeDtypeStruct((DIM, 2 * BATCH), jnp.float32),
    input_output_aliases={1: 0},
    scratch_shapes=[pltpu.VMEM((DIM, DIM), jnp.float32)],
)


def kernel(emb_src, emb_tgt, W_enc, b_enc, W_dec, src_id, tgt_id):
    sidx = src_id.astype(jnp.int32).reshape(NW, ROWS_PER_W)
    tidx = tgt_id.astype(jnp.int32).reshape(NW, ROWS_PER_W)
    gather = _sc_gather_fn()
    t_src = _tf_src(jnp.transpose(emb_src), W_enc, W_dec,
                    b_enc.reshape(1, DIM))
    src_rows = gather(t_src, sidx)
    t_tgt = _tf_tgt(jnp.transpose(emb_tgt))
    tgt_rows = gather(t_tgt, tidx)
    out_t = _trans_b(tgt_rows, _trans_a(src_rows))
    return jnp.transpose(out_t)
